# baseline (speedup 1.0000x reference)
import functools

import jax
import jax.numpy as jnp
from jax import lax
from jax.experimental import pallas as pl
from jax.experimental.pallas import tpu as pltpu
from jax.experimental.pallas import tpu_sc as plsc

B, N, D = 4, 4096, 2048
NUM_BRANCHES, F, MAX_INST = 8, 8, 512
TOK = B * N
NC, NS, L = 2, 16, 16
TH = NC
CS = NS
CW = D // CS
TPW = TOK // TH
NROWS = NUM_BRANCHES * F
TABR = NROWS + 8
IPR = MAX_INST + 8
BT = 32
NB = TPW // BT
ICH = 1024


def _prep_body(te_ref, fe_ref, ve_ref, tab_ref):
    a = te_ref[...] + fe_ref[...]
    top = jnp.concatenate([a, ve_ref[...]], axis=1)
    pad = jnp.zeros((TABR - NROWS, 2 * D), jnp.float32)
    tab_ref[...] = jnp.concatenate([top, pad], axis=0)


def _sc_body(tab_hbm, ip_hbm, t_hbm, f_hbm, i_hbm, p_hbm, v_hbm, out_hbm,
             tab_v, ip_v, cidx_v, iidx_v, vals_v, tt, tf, ti, tp,
             st0, st1, sem0, sem1):
    th = lax.axis_index("c")
    cs = lax.axis_index("s")

    pltpu.sync_copy(tab_hbm.at[cs], tab_v)
    pltpu.sync_copy(ip_hbm.at[cs], ip_v)
    pltpu.sync_copy(v_hbm.at[th], vals_v)

    for ch in range(TPW // ICH):
        sl = pl.ds(ch * ICH, ICH)
        pltpu.sync_copy(t_hbm.at[th, sl], tt)
        pltpu.sync_copy(f_hbm.at[th, sl], tf)
        pltpu.sync_copy(i_hbm.at[th, sl], ti)
        pltpu.sync_copy(p_hbm.at[th, sl], tp)
        for kk in range(ICH // L):
            s = pl.ds(kk * L, L)
            t = tt[s]
            f = tf[s]
            ii = ti[s]
            p = tp[s]
            loc = jnp.clip(f - t * F, 0, F - 1)
            c = t * F + loc
            masked = p != 0
            dst = pl.ds(ch * ICH + kk * L, L)
            cidx_v[dst] = jnp.where(masked, NROWS, c) * (2 * CW)
            iidx_v[dst] = jnp.where(masked, MAX_INST, ii) * CW

    lane = lax.iota(jnp.int32, L)
    tokbase = th * TPW
    colbase = cs * CW

    def batch(m, stage, sem, drain):
        if drain:
            pltpu.make_async_copy(
                stage, out_hbm.at[pl.ds(0, BT), pl.ds(0, CW)], sem).wait()
        for gl in range(BT // L):
            goff = m * BT + gl * L
            ca = cidx_v[pl.ds(goff, L)]
            ia = iidx_v[pl.ds(goff, L)]
            vv = vals_v[pl.ds(goff, L)]
            cw = ca + CW
            rvec = lane + (gl * L)

            def col(d):
                dsp = jnp.full((L,), d, jnp.int32)
                a = plsc.load_gather(tab_v, [ca + dsp])
                w = plsc.load_gather(tab_v, [cw + dsp])
                pv = plsc.load_gather(ip_v, [ia + dsp])
                r = pv + a + vv * w
                plsc.store_scatter(stage, [rvec, dsp], r)

            pl.loop(0, CW, unroll=8)(col)
        dst = out_hbm.at[pl.ds(tokbase + m * BT, BT), pl.ds(colbase, CW)]
        pltpu.async_copy(stage, dst, sem)

    batch(0, st0, sem0, False)
    batch(1, st1, sem1, False)

    def pair(pp):
        batch(pp * 2, st0, sem0, True)
        batch(pp * 2 + 1, st1, sem1, True)

    pl.loop(1, NB // 2)(pair)

    pltpu.make_async_copy(st0, out_hbm.at[pl.ds(0, BT), pl.ds(0, CW)],
                          sem0).wait()
    pltpu.make_async_copy(st1, out_hbm.at[pl.ds(0, BT), pl.ds(0, CW)],
                          sem1).wait()


@jax.jit
def _run(type_ids, inst_ids, field_ids, values, padding_mask,
         type_emb, inst_pos, field_emb, value_emb):
    te64 = jnp.repeat(type_emb, F, axis=0)
    fe = field_emb.reshape(NROWS, D)
    ve = value_emb.reshape(NROWS, D)

    tab = pl.pallas_call(
        _prep_body,
        out_shape=jax.ShapeDtypeStruct((TABR, 2 * D), jnp.float32),
    )(te64, fe, ve)

    tabA = tab[:, :D].reshape(TABR, CS, CW)
    tabV = tab[:, D:].reshape(TABR, CS, CW)
    tab4 = jnp.concatenate([tabA, tabV], axis=2)
    tab4 = tab4.transpose(1, 0, 2).reshape(CS, TABR * 2 * CW)

    ip = jnp.pad(inst_pos, ((0, IPR - MAX_INST), (0, 0)))
    ip4 = ip.reshape(IPR, CS, CW).transpose(1, 0, 2).reshape(CS, IPR * CW)

    t2 = type_ids.reshape(TH, TPW).astype(jnp.int32)
    f2 = field_ids.reshape(TH, TPW).astype(jnp.int32)
    i2 = inst_ids.reshape(TH, TPW).astype(jnp.int32)
    p2 = padding_mask.reshape(TH, TPW).astype(jnp.int32)
    v2 = values.reshape(TH, TPW)

    mesh = plsc.VectorSubcoreMesh(core_axis_name="c", subcore_axis_name="s")
    out = pl.kernel(
        _sc_body,
        out_type=jax.ShapeDtypeStruct((TOK, D), jnp.float32),
        mesh=mesh,
        compiler_params=pltpu.CompilerParams(needs_layout_passes=False),
        scratch_types=[
            pltpu.VMEM((TABR * 2 * CW,), jnp.float32),
            pltpu.VMEM((IPR * CW,), jnp.float32),
            pltpu.VMEM((TPW,), jnp.int32),
            pltpu.VMEM((TPW,), jnp.int32),
            pltpu.VMEM((TPW,), jnp.float32),
            pltpu.VMEM((ICH,), jnp.int32),
            pltpu.VMEM((ICH,), jnp.int32),
            pltpu.VMEM((ICH,), jnp.int32),
            pltpu.VMEM((ICH,), jnp.int32),
            pltpu.VMEM((BT, CW), jnp.float32),
            pltpu.VMEM((BT, CW), jnp.float32),
            pltpu.SemaphoreType.DMA,
            pltpu.SemaphoreType.DMA,
        ],
    )(tab4, ip4, t2, f2, i2, p2, v2)
    return out.reshape(B, N, D)


def kernel(type_ids, inst_ids, field_ids, values, padding_mask,
           type_emb, inst_pos, field_emb, value_emb):
    return _run(type_ids, inst_ids, field_ids, values, padding_mask,
                type_emb, inst_pos, field_emb, value_emb)
